# baseline (speedup 1.0000x reference)
import functools

import jax
import jax.numpy as jnp
from jax.experimental import pallas as pl

_TL_TILE = 256


def _scorer_kernel(test_ref, sup_ref, tgt_ref, out_ref, proto_ref):
    s, tile, d = test_ref.shape[1], test_ref.shape[2], test_ref.shape[3]
    sl = sup_ref.shape[2]
    t = tgt_ref.shape[3]
    n = s * sl

    tm = jnp.mean(test_ref[0], axis=0)

    sup = sup_ref[0].reshape(n, d)
    tgt = tgt_ref[0].reshape(n, t)

    t2 = jnp.sum(tm * tm, axis=1, keepdims=True)
    s2 = jnp.sum(sup * sup, axis=1, keepdims=True)
    dot = jax.lax.dot_general(
        tm, sup, (((1,), (1,)), ((), ())),
        preferred_element_type=jnp.float32,
    )
    d2 = jnp.maximum(t2 + s2.T - 2.0 * dot, 0.0)

    minval = jnp.min(d2, axis=1, keepdims=True)
    iota = jax.lax.broadcasted_iota(jnp.int32, (tile, n), 1)
    idx = jnp.min(jnp.where(d2 == minval, iota, n), axis=1, keepdims=True)
    onehot = (iota == idx).astype(jnp.float32)

    sim = jax.lax.dot_general(
        onehot, tgt, (((1,), (0,)), ((), ())),
        preferred_element_type=jnp.float32,
    )

    psum = jax.lax.dot_general(
        tgt, sup, (((0,), (0,)), ((), ())),
        preferred_element_type=jnp.float32,
    )
    cnt = jnp.sum(tgt, axis=0, keepdims=True)
    proto = psum / (cnt.T + 0.0001)

    sim1 = jax.lax.dot_general(
        tm, proto, (((1,), (1,)), ((), ())),
        preferred_element_type=jnp.float32,
    )

    out_ref[0] = sim + 0.5 * sim1
    proto_ref[0] = proto


@functools.partial(jax.jit, static_argnames=())
def kernel(test_reps, support_reps, test_output_mask, support_output_mask, support_targets):
    del test_output_mask, support_output_mask
    b, s, tl, d = test_reps.shape
    sl = support_reps.shape[2]
    t = support_targets.shape[3]
    tiles = tl // _TL_TILE

    out, proto = pl.pallas_call(
        _scorer_kernel,
        grid=(b, tiles),
        in_specs=[
            pl.BlockSpec((1, s, _TL_TILE, d), lambda i, j: (i, 0, j, 0)),
            pl.BlockSpec((1, s, sl, d), lambda i, j: (i, 0, 0, 0)),
            pl.BlockSpec((1, s, sl, t), lambda i, j: (i, 0, 0, 0)),
        ],
        out_specs=[
            pl.BlockSpec((1, _TL_TILE, t), lambda i, j: (i, j, 0)),
            pl.BlockSpec((1, t, d), lambda i, j: (i, 0, 0)),
        ],
        out_shape=[
            jax.ShapeDtypeStruct((b, tl, t), jnp.float32),
            jax.ShapeDtypeStruct((b, t, d), jnp.float32),
        ],
    )(test_reps, support_reps, support_targets)
    return (out, proto)


# speedup vs baseline: 1.1937x; 1.1937x over previous
import functools

import jax
import jax.numpy as jnp
from jax.experimental import pallas as pl


def _scorer_kernel(test_ref, sup_ref, tgt_ref, out_ref, proto_ref):
    s, tl, d = test_ref.shape[1], test_ref.shape[2], test_ref.shape[3]
    sl = sup_ref.shape[2]
    t = tgt_ref.shape[3]
    n = s * sl

    tm = jnp.mean(test_ref[0], axis=0)

    sup = sup_ref[0].reshape(n, d)
    tgt = tgt_ref[0].reshape(n, t)

    tag_iota = jax.lax.broadcasted_iota(jnp.int32, (n, t), 1).astype(jnp.float32)
    labels_col = jnp.sum(tgt * tag_iota, axis=1, keepdims=True)
    labels_row = labels_col.reshape(1, n).astype(jnp.int32)

    t2 = jnp.sum(tm * tm, axis=1, keepdims=True)
    s2 = jnp.sum(sup * sup, axis=1, keepdims=True)
    dot = jax.lax.dot_general(
        tm, sup, (((1,), (1,)), ((), ())),
        preferred_element_type=jnp.float32,
    )
    d2 = jnp.maximum(t2 + s2.reshape(1, n) - 2.0 * dot, 0.0)

    minval = jnp.min(d2, axis=1, keepdims=True)
    iota = jax.lax.broadcasted_iota(jnp.int32, (tl, n), 1)
    key = iota * 64 + labels_row
    win = jnp.min(jnp.where(d2 == minval, key, n * 64), axis=1, keepdims=True)
    win_label = jax.lax.rem(win, 64)

    out_iota = jax.lax.broadcasted_iota(jnp.int32, (tl, t), 1)
    sim = (out_iota == win_label).astype(jnp.float32)

    psum = jax.lax.dot_general(
        tgt, sup, (((0,), (0,)), ((), ())),
        preferred_element_type=jnp.float32,
    )
    cnt = jnp.sum(tgt, axis=0, keepdims=True)
    proto = psum / (cnt.reshape(t, 1) + 0.0001)

    sim1 = jax.lax.dot_general(
        tm, proto, (((1,), (1,)), ((), ())),
        preferred_element_type=jnp.float32,
    )

    out_ref[0] = sim + 0.5 * sim1
    proto_ref[0] = proto


@functools.partial(jax.jit, static_argnames=())
def kernel(test_reps, support_reps, test_output_mask, support_output_mask, support_targets):
    del test_output_mask, support_output_mask
    b, s, tl, d = test_reps.shape
    sl = support_reps.shape[2]
    t = support_targets.shape[3]

    out, proto = pl.pallas_call(
        _scorer_kernel,
        grid=(b,),
        in_specs=[
            pl.BlockSpec((1, s, tl, d), lambda i: (i, 0, 0, 0)),
            pl.BlockSpec((1, s, sl, d), lambda i: (i, 0, 0, 0)),
            pl.BlockSpec((1, s, sl, t), lambda i: (i, 0, 0, 0)),
        ],
        out_specs=[
            pl.BlockSpec((1, tl, t), lambda i: (i, 0, 0)),
            pl.BlockSpec((1, t, d), lambda i: (i, 0, 0)),
        ],
        out_shape=[
            jax.ShapeDtypeStruct((b, tl, t), jnp.float32),
            jax.ShapeDtypeStruct((b, t, d), jnp.float32),
        ],
    )(test_reps, support_reps, support_targets)
    return (out, proto)
